# baseline (speedup 1.0000x reference)
import jax
import jax.numpy as jnp
import numpy as np
from jax import lax
from jax.experimental import pallas as pl
from jax.experimental.pallas import tpu as pltpu

B = 128
N_PER = 32
NUM_RADIAL = 128
EMB_ATOM = 128
EMB_EDGE = 128
LATENT = 256
NUM_BLOCKS = 3
CUTOFF = 6.0
NUM_TYPES = 100
P_EXP = 5

GROUP = 16

_WIDTH = CUTOFF / (NUM_RADIAL - 1)
_S = float(np.sqrt(0.5 * np.log2(np.e)) / _WIDTH)


def _silu_h(y):
    return y * jnp.tanh(y) + y


def _body(types_ref, frac_ref, fracT_ref, cell_ref, t_ref, atom_emb_ref,
          W_t_ref, W_edge_ref, b_edge_ref, W_rbf_ref, W_m_ref, W_e2_ref,
          W_h_ref, W_o1_ref, W_o2_ref, out_ref):
    n = N_PER
    G = GROUP
    F = EMB_EDGE
    R = NUM_RADIAL

    types = types_ref[0]
    frac = frac_ref[0].reshape(G, n, 3)
    fracT = fracT_ref[0].reshape(G, 3, n)
    cell = cell_ref[...]
    tmat = t_ref[0]

    pos = lax.dot_general(frac, cell, (((2,), (1,)), ((0,), (0,))),
                          preferred_element_type=jnp.float32)
    posT = lax.dot_general(cell, fracT, (((1,), (1,)), ((0,), (0,))),
                           preferred_element_type=jnp.float32)
    dist2 = jnp.zeros((G, n, n), jnp.float32) + 1e-9
    for k in range(3):
        delta = posT[:, k:k + 1, :] - pos[:, :, k:k + 1]
        dist2 = dist2 + delta * delta
    dist = jnp.sqrt(dist2)
    d = dist * (1.0 / CUTOFF)
    d2 = d * d
    d5 = d2 * d2 * d
    env = 1.0 - 21.0 * d5 + 35.0 * (d5 * d) - 15.0 * (d5 * d2)
    ii = lax.broadcasted_iota(jnp.int32, (G, n, n), 1)
    jj = lax.broadcasted_iota(jnp.int32, (G, n, n), 2)
    env = jnp.where((d < 1.0) & (ii != jj), env, 0.0)

    ck = (lax.broadcasted_iota(jnp.int32, (1, R), 1)
          .astype(jnp.float32) * (_S * _WIDTH)).reshape(1, 1, 1, R)
    sdist = dist * _S
    sd4 = jnp.broadcast_to(sdist.reshape(G, n, n, 1), (G, n, n, R))
    env4 = jnp.broadcast_to(env.reshape(G, n, n, 1), (G, n, n, R))
    z = sd4 - ck
    rbf2 = (jnp.exp2(-(z * z)) * env4).reshape(G * n * n, R)

    oh = (types == lax.broadcasted_iota(jnp.int32, (G * n, NUM_TYPES), 1)
          ).astype(jnp.float32)
    tw = _silu_h(jnp.dot(tmat, W_t_ref[...],
                       preferred_element_type=jnp.float32))
    h = (jnp.dot(oh, atom_emb_ref[...], preferred_element_type=jnp.float32)
         .reshape(G, n, EMB_ATOM) + tw.reshape(G, 1, EMB_ATOM)
         ).reshape(G * n, EMB_ATOM)

    W_edge = W_edge_ref[...]
    hw1 = (jnp.dot(h, W_edge[:F], preferred_element_type=jnp.float32)
           + b_edge_ref[...])
    hw2 = jnp.dot(h, W_edge[F:2 * F], preferred_element_type=jnp.float32)
    rw = jnp.dot(rbf2, W_edge[2 * F:],
                 preferred_element_type=jnp.float32)
    e = _silu_h(hw1.reshape(G, n, 1, F) + hw2.reshape(G, 1, n, F)
              + rw.reshape(G, n, n, F)).reshape(G * n * n, F)

    for blk in range(NUM_BLOCKS):
        rb = jnp.dot(rbf2, W_rbf_ref[blk], preferred_element_type=jnp.float32)
        m = _silu_h(jnp.dot(e, W_m_ref[blk],
                          preferred_element_type=jnp.float32)) * rb
        agg = jnp.sum(m.reshape(G, n, n, F), axis=1).reshape(G * n, F)
        h = h + _silu_h(jnp.dot(agg, W_h_ref[blk],
                              preferred_element_type=jnp.float32))
        e = e + _silu_h(jnp.dot(e, W_e2_ref[blk],
                              preferred_element_type=jnp.float32)) * rb

    eps = jnp.dot(_silu_h(jnp.dot(h, W_o1_ref[...],
                                preferred_element_type=jnp.float32)),
                  W_o2_ref[...], preferred_element_type=jnp.float32)
    out_ref[...] = jnp.sum(eps.reshape(G, n, 1), axis=1, keepdims=True)


def kernel(t, atom_types, frac_coords, lattices_rep, num_atoms, node2graph,
           lattices_mat, atom_emb, W_t, W_edge, b_edge, W_rbf_blocks,
           W_m_blocks, W_h_blocks, W_e2_blocks, W_o1, W_o2):
    S = B // GROUP
    types3 = atom_types.reshape(S, GROUP * N_PER, 1)
    frac4 = frac_coords.reshape(S, GROUP * N_PER, 3)
    fracT4 = (frac_coords.reshape(B, N_PER, 3).transpose(0, 2, 1)
              .reshape(S, GROUP * 3, N_PER))
    t3 = t.reshape(S, GROUP, LATENT)
    b_edge2 = 0.5 * b_edge.reshape(1, EMB_EDGE)

    def full(shape):
        return pl.BlockSpec(shape, lambda g: (0,) * len(shape))

    out = pl.pallas_call(
        _body,
        grid=(S,),
        in_specs=[
            pl.BlockSpec((1, GROUP * N_PER, 1), lambda g: (g, 0, 0)),
            pl.BlockSpec((1, GROUP * N_PER, 3), lambda g: (g, 0, 0)),
            pl.BlockSpec((1, GROUP * 3, N_PER), lambda g: (g, 0, 0)),
            pl.BlockSpec((GROUP, 3, 3), lambda g: (g, 0, 0)),
            pl.BlockSpec((1, GROUP, LATENT), lambda g: (g, 0, 0)),
            full((NUM_TYPES, EMB_ATOM)),
            full((LATENT, EMB_ATOM)),
            full((2 * EMB_ATOM + NUM_RADIAL, EMB_EDGE)),
            full((1, EMB_EDGE)),
            full((NUM_BLOCKS, NUM_RADIAL, EMB_EDGE)),
            full((NUM_BLOCKS, EMB_EDGE, EMB_EDGE)),
            full((NUM_BLOCKS, EMB_EDGE, EMB_EDGE)),
            full((NUM_BLOCKS, EMB_EDGE, EMB_ATOM)),
            full((EMB_ATOM, 64)),
            full((64, 1)),
        ],
        out_specs=pl.BlockSpec((GROUP, 1, 1), lambda g: (g, 0, 0)),
        out_shape=jax.ShapeDtypeStruct((B, 1, 1), jnp.float32),
        compiler_params=pltpu.CompilerParams(
            dimension_semantics=("parallel",)),
    )(types3, frac4, fracT4, lattices_mat, t3, atom_emb, 0.5 * W_t,
      0.5 * W_edge, b_edge2, W_rbf_blocks, 0.5 * W_m_blocks,
      0.5 * W_e2_blocks, 0.5 * W_h_blocks, 0.5 * W_o1, W_o2)
    return out.reshape(B, 1)


# speedup vs baseline: 1.0282x; 1.0282x over previous
import jax
import jax.numpy as jnp
import numpy as np
from jax import lax
from jax.experimental import pallas as pl
from jax.experimental.pallas import tpu as pltpu

B = 128
N_PER = 32
NUM_RADIAL = 128
EMB_ATOM = 128
EMB_EDGE = 128
LATENT = 256
NUM_BLOCKS = 3
CUTOFF = 6.0
NUM_TYPES = 100
P_EXP = 5

GROUP = 16

_WIDTH = CUTOFF / (NUM_RADIAL - 1)
_S = float(np.sqrt(0.5 * np.log2(np.e)) / _WIDTH)


def _silu_h(y):
    return y * jnp.tanh(y) + y


def _silu(x):
    return _silu_h(0.5 * x)


def _body(types_ref, frac_ref, cell_ref, t_ref, atom_emb_ref,
          W_t_ref, W_edge_ref, b_edge_ref, W_rbf_ref, W_m_ref, W_e2_ref,
          W_h_ref, W_o1_ref, W_o2_ref, out_ref):
    n = N_PER
    G = GROUP
    F = EMB_EDGE
    R = NUM_RADIAL

    types = types_ref[0]
    frac = frac_ref[0].reshape(G, n, 3)
    cell = cell_ref[...]
    tmat = t_ref[0]

    pos = lax.dot_general(frac, cell, (((2,), (1,)), ((0,), (0,))),
                          preferred_element_type=jnp.float32)
    dist2 = jnp.zeros((G, n, n), jnp.float32)
    for k in range(3):
        pk = pos[:, :, k]
        dk = pk.reshape(G, n, 1) - pk.reshape(G, 1, n)
        dist2 = dist2 + dk * dk
    dist = jnp.sqrt(jnp.maximum(dist2, 1e-9))
    d = dist * (1.0 / CUTOFF)
    d2 = d * d
    d5 = d2 * d2 * d
    env = 1.0 - 21.0 * d5 + 35.0 * (d5 * d) - 15.0 * (d5 * d2)
    ii = lax.broadcasted_iota(jnp.int32, (G, n, n), 1)
    jj = lax.broadcasted_iota(jnp.int32, (G, n, n), 2)
    env = jnp.where((d < 1.0) & (ii != jj), env, 0.0)

    ck = (lax.broadcasted_iota(jnp.int32, (1, R), 1)
          .astype(jnp.float32) * (_S * _WIDTH)).reshape(1, 1, 1, R)
    sdist = dist * _S
    sd4 = jnp.broadcast_to(sdist.reshape(G, n, n, 1), (G, n, n, R))
    env4 = jnp.broadcast_to(env.reshape(G, n, n, 1), (G, n, n, R))
    z = sd4 - ck
    rbf2 = (jnp.exp2(-(z * z)) * env4).reshape(G * n * n, R)

    oh = (types == lax.broadcasted_iota(jnp.int32, (G * n, NUM_TYPES), 1)
          ).astype(jnp.float32)
    tw = _silu(jnp.dot(tmat, W_t_ref[...],
                       preferred_element_type=jnp.float32))
    h = (jnp.dot(oh, atom_emb_ref[...], preferred_element_type=jnp.float32)
         .reshape(G, n, EMB_ATOM) + tw.reshape(G, 1, EMB_ATOM)
         ).reshape(G * n, EMB_ATOM)

    W_edge = W_edge_ref[...]
    hw1 = (jnp.dot(h, W_edge[:F], preferred_element_type=jnp.float32)
           + b_edge_ref[...])
    hw2 = jnp.dot(h, W_edge[F:2 * F], preferred_element_type=jnp.float32)
    rw = jnp.dot(rbf2, W_edge[2 * F:],
                 preferred_element_type=jnp.float32)
    e = _silu(hw1.reshape(G, n, 1, F) + hw2.reshape(G, 1, n, F)
              + rw.reshape(G, n, n, F)).reshape(G * n * n, F)

    for blk in range(NUM_BLOCKS):
        rb = jnp.dot(rbf2, W_rbf_ref[blk], preferred_element_type=jnp.float32)
        m = _silu(jnp.dot(e, W_m_ref[blk],
                          preferred_element_type=jnp.float32)) * rb
        agg = jnp.sum(m.reshape(G, n, n, F), axis=1).reshape(G * n, F)
        h = h + _silu(jnp.dot(agg, W_h_ref[blk],
                              preferred_element_type=jnp.float32))
        e = e + _silu(jnp.dot(e, W_e2_ref[blk],
                              preferred_element_type=jnp.float32)) * rb

    eps = jnp.dot(_silu(jnp.dot(h, W_o1_ref[...],
                                preferred_element_type=jnp.float32)),
                  W_o2_ref[...], preferred_element_type=jnp.float32)
    out_ref[...] = jnp.sum(eps.reshape(G, n, 1), axis=1, keepdims=True)


def kernel(t, atom_types, frac_coords, lattices_rep, num_atoms, node2graph,
           lattices_mat, atom_emb, W_t, W_edge, b_edge, W_rbf_blocks,
           W_m_blocks, W_h_blocks, W_e2_blocks, W_o1, W_o2):
    S = B // GROUP
    types3 = atom_types.reshape(S, GROUP * N_PER, 1)
    frac4 = frac_coords.reshape(S, GROUP * N_PER, 3)
    t3 = t.reshape(S, GROUP, LATENT)
    b_edge2 = b_edge.reshape(1, EMB_EDGE)

    def full(shape):
        return pl.BlockSpec(shape, lambda g: (0,) * len(shape))

    out = pl.pallas_call(
        _body,
        grid=(S,),
        in_specs=[
            pl.BlockSpec((1, GROUP * N_PER, 1), lambda g: (g, 0, 0)),
            pl.BlockSpec((1, GROUP * N_PER, 3), lambda g: (g, 0, 0)),
            pl.BlockSpec((GROUP, 3, 3), lambda g: (g, 0, 0)),
            pl.BlockSpec((1, GROUP, LATENT), lambda g: (g, 0, 0)),
            full((NUM_TYPES, EMB_ATOM)),
            full((LATENT, EMB_ATOM)),
            full((2 * EMB_ATOM + NUM_RADIAL, EMB_EDGE)),
            full((1, EMB_EDGE)),
            full((NUM_BLOCKS, NUM_RADIAL, EMB_EDGE)),
            full((NUM_BLOCKS, EMB_EDGE, EMB_EDGE)),
            full((NUM_BLOCKS, EMB_EDGE, EMB_EDGE)),
            full((NUM_BLOCKS, EMB_EDGE, EMB_ATOM)),
            full((EMB_ATOM, 64)),
            full((64, 1)),
        ],
        out_specs=pl.BlockSpec((GROUP, 1, 1), lambda g: (g, 0, 0)),
        out_shape=jax.ShapeDtypeStruct((B, 1, 1), jnp.float32),
        compiler_params=pltpu.CompilerParams(
            dimension_semantics=("parallel",)),
    )(types3, frac4, lattices_mat, t3, atom_emb, W_t, W_edge,
      b_edge2, W_rbf_blocks, W_m_blocks, W_e2_blocks, W_h_blocks, W_o1, W_o2)
    return out.reshape(B, 1)
